# baseline (speedup 1.0000x reference)
import functools

import jax
import jax.numpy as jnp
from jax import lax
from jax.experimental import pallas as pl
from jax.experimental.pallas import tpu as pltpu
from jax.experimental.pallas import tpu_sc as plsc

EMBED_DIM = 64
BATCH = 4096
SEQ_LEN = 200
TOTAL = BATCH * SEQ_LEN

_info = plsc.get_sparse_core_info()
NC, NS = _info.num_cores, _info.num_subcores
NW = NC * NS
BPW = TOTAL // NW

CHUNK = 512
SUB = 128
NSUB = CHUNK // SUB
NCHUNK = BPW // CHUNK

_mesh = plsc.VectorSubcoreMesh(core_axis_name="c", subcore_axis_name="s")


@functools.partial(
    pl.kernel,
    mesh=_mesh,
    out_type=jax.ShapeDtypeStruct((TOTAL, EMBED_DIM), jnp.float32),
    scratch_types=[
        pltpu.VMEM((CHUNK,), jnp.int32),
        pltpu.VMEM((CHUNK, EMBED_DIM), jnp.float32),
        pltpu.SemaphoreType.DMA,
    ],
    compiler_params=pltpu.CompilerParams(use_tc_tiling_on_sc=False),
)
def _gather_rows(idx_hbm, table_hbm, out_hbm, idx_v, rows_v, sem):
    wid = lax.axis_index("s") * NC + lax.axis_index("c")
    base = wid * BPW

    def body(g, carry):
        off = base + g * CHUNK
        pltpu.sync_copy(idx_hbm.at[pl.ds(off, CHUNK)], idx_v)
        copies = []
        for j in range(NSUB):
            copies.append(
                pltpu.async_copy(
                    table_hbm.at[idx_v.at[pl.ds(j * SUB, SUB)]],
                    rows_v.at[pl.ds(j * SUB, SUB)],
                    sem,
                )
            )
        for c in copies:
            c.wait()
        pltpu.sync_copy(rows_v, out_hbm.at[pl.ds(off, CHUNK)])
        return carry

    lax.fori_loop(0, NCHUNK, body, 0)


def kernel(input_ids, table):
    flat = input_ids.reshape(-1).astype(jnp.int32)
    out = _gather_rows(flat, table)
    return out.reshape(input_ids.shape + (EMBED_DIM,))


# speedup vs baseline: 1.0392x; 1.0392x over previous
import functools

import jax
import jax.numpy as jnp
from jax import lax
from jax.experimental import pallas as pl
from jax.experimental.pallas import tpu as pltpu
from jax.experimental.pallas import tpu_sc as plsc

EMBED_DIM = 64
TOTAL = 4096 * 200

_info = plsc.get_sparse_core_info()
NC, NS = _info.num_cores, _info.num_subcores
NW = NC * NS
BPW = TOTAL // NW

SUB = 128
CHUNK = 256
NSUB = CHUNK // SUB
NB = 4
NCHUNK = BPW // CHUNK
NBODY = NCHUNK // NB

_mesh = plsc.VectorSubcoreMesh(core_axis_name="c", subcore_axis_name="s")


@functools.partial(
    pl.kernel,
    mesh=_mesh,
    out_type=jax.ShapeDtypeStruct((TOTAL, EMBED_DIM), jnp.float32),
    scratch_types=[
        pltpu.VMEM((BPW,), jnp.int32),
        pltpu.VMEM((NB, CHUNK, EMBED_DIM), jnp.float32),
        pltpu.SemaphoreType.DMA((NB,)),
        pltpu.SemaphoreType.DMA((NB,)),
    ],
    compiler_params=pltpu.CompilerParams(use_tc_tiling_on_sc=False),
)
def _gather_rows(idx_hbm, table_hbm, out_hbm, idx_all, rows, gsem, ssem):
    wid = lax.axis_index("s") * NC + lax.axis_index("c")
    base = wid * BPW
    pltpu.sync_copy(idx_hbm.at[pl.ds(base, BPW)], idx_all)

    def body(k, carry):
        gathers = []
        for b in range(NB):
            off = (k * NB + b) * CHUNK

            @pl.when(k > 0)
            def _():
                pltpu.make_async_copy(
                    rows.at[b], out_hbm.at[pl.ds(base + off, CHUNK)], ssem.at[b]
                ).wait()

            for j in range(NSUB):
                gathers.append(
                    pltpu.async_copy(
                        table_hbm.at[idx_all.at[pl.ds(off + j * SUB, SUB)]],
                        rows.at[b, pl.ds(j * SUB, SUB)],
                        gsem.at[b],
                    )
                )
        for b in range(NB):
            off = (k * NB + b) * CHUNK
            for j in range(NSUB):
                gathers[b * NSUB + j].wait()
            pltpu.async_copy(rows.at[b], out_hbm.at[pl.ds(base + off, CHUNK)], ssem.at[b])
        return carry

    lax.fori_loop(0, NBODY, body, 0)

    for b in range(NB):
        off = ((NBODY - 1) * NB + b) * CHUNK
        pltpu.make_async_copy(
            rows.at[b], out_hbm.at[pl.ds(base + off, CHUNK)], ssem.at[b]
        ).wait()


def kernel(input_ids, table):
    flat = input_ids.reshape(-1).astype(jnp.int32)
    out = _gather_rows(flat, table)
    return out.reshape(input_ids.shape + (EMBED_DIM,))
